# Initial kernel scaffold; baseline (speedup 1.0000x reference)
import jax
import jax.numpy as jnp
from jax.experimental import pallas as pl


def kernel(points):
    raise NotImplementedError("write your pallas kernel here")


# speedup vs baseline: 1.2890x; 1.2890x over previous
import functools

import jax
import jax.numpy as jnp
from jax import lax
from jax.experimental import pallas as pl
from jax.experimental.pallas import tpu as pltpu
from jax.experimental.pallas import tpu_sc as plsc

B, T = 1024, 256
NUM_SLICES = 20
N_SIDE = 21
NPTS = 42
ROW_W = NPTS * 2

NC, NS, L = 2, 16, 16
NW = NC * NS
ROWS = B * T
ROWS_PER_W = ROWS // NW
CH = 512
NCHUNK = ROWS_PER_W // CH
NBLK = CH // L

import numpy as _np

_TARGETS = _np.linspace(0.05, 0.95, NUM_SLICES, dtype=_np.float32)
_T0 = float(_TARGETS[0])
_INV_STEP = float(1.0 / (_TARGETS[1] - _TARGETS[0]))
_VOL_C = float(3.14159 / 4.0 / NUM_SLICES)


def _oddeven_network(n):
    p2 = 1 << (n - 1).bit_length()
    pairs = []

    def merge(lo, hi, r):
        step = r * 2
        if step < hi - lo:
            merge(lo, hi, step)
            merge(lo + r, hi, step)
            for i in range(lo + r, hi - r, step):
                pairs.append((i, i + r))
        else:
            pairs.append((lo, lo + r))

    def sort(lo, hi):
        if hi - lo >= 2:
            mid = lo + (hi - lo) // 2
            sort(lo, mid)
            sort(mid, hi)
            merge(lo, hi, 1)

    sort(0, p2)
    return [(a, b) for (a, b) in pairs if a < n and b < n]


_NET = _oddeven_network(N_SIDE)


def _side_interp(hs, rs, lane, shb, srb, hist):
    hs = list(hs)
    rs = list(rs)
    for a, b in _NET:
        swap = hs[a] > hs[b]
        ha = jnp.minimum(hs[a], hs[b])
        hb = jnp.maximum(hs[a], hs[b])
        ra = jnp.where(swap, rs[b], rs[a])
        rb = jnp.where(swap, rs[a], rs[b])
        hs[a], hs[b], rs[a], rs[b] = ha, hb, ra, rb

    zero_i = jnp.zeros((L,), jnp.int32)
    one_i = jnp.ones((L,), jnp.int32)
    for j in range(N_SIDE):
        shb[j, :] = hs[j]
        srb[j, :] = rs[j]
        hist[j, :] = zero_i
    for j in range(N_SIDE):
        q = (hs[j] - _T0) * _INV_STEP + 1.0
        q = jnp.minimum(jnp.maximum(q, 0.0), float(NUM_SLICES) + 0.99)
        plsc.addupdate_scatter(hist, [q.astype(jnp.int32), lane], one_i)

    out = []
    cnt = zero_i
    for k in range(NUM_SLICES):
        cnt = cnt + hist[k, :]
        lo = jnp.minimum(jnp.maximum(cnt - 1, 0), N_SIDE - 2)
        hi = lo + 1
        xl = plsc.load_gather(shb, [lo, lane])
        xh = plsc.load_gather(shb, [hi, lane])
        yl = plsc.load_gather(srb, [lo, lane])
        yh = plsc.load_gather(srb, [hi, lane])
        w = (float(_TARGETS[k]) - xl) / (xh - xl + 1e-6)
        w = jnp.minimum(jnp.maximum(w, 0.0), 1.0)
        out.append(yl + w * (yh - yl))
    return out


def _sc_body(pts_hbm, out_hbm, buf, shb, srb, hist, r1buf, outbuf):
    wid = lax.axis_index("s") * NC + lax.axis_index("c")
    row0 = wid * ROWS_PER_W
    lane = lax.iota(jnp.int32, L)

    def chunk_body(ci, carry):
        base = row0 + ci * CH
        pltpu.sync_copy(pts_hbm.at[pl.ds(base * ROW_W, CH * ROW_W)], buf)

        def block_body(bi, c2):
            roff = bi * (L * ROW_W) + lane * ROW_W

            def gcol(col):
                return plsc.load_gather(buf, [roff + col])

            p0x, p0y = gcol(0), gcol(1)
            q0x, q0y = gcol(82), gcol(83)
            a0x, a0y = gcol(40), gcol(41)
            a1x, a1y = gcol(42), gcol(43)
            bx = (p0x + q0x) * 0.5
            by = (p0y + q0y) * 0.5
            ax = (a0x + a1x) * 0.5
            ay = (a0y + a1y) * 0.5
            dx = ax - bx
            dy = ay - by
            len2 = jnp.maximum(dx * dx + dy * dy, 1e-20)
            yi = 0x5F3759DF - (plsc.bitcast(len2, jnp.int32) >> 1)
            y = plsc.bitcast(yi, jnp.float32)
            hx = 0.5 * len2
            for _ in range(3):
                y = y * (1.5 - hx * y * y)
            length = len2 * y
            safe = jnp.maximum(length, 1e-6)
            inv_s = 1.0 / safe
            scale = inv_s * inv_s

            r2k = None
            acc = jnp.zeros((L,), jnp.float32)
            for side in (0, 1):
                hs = []
                rs = []
                jbase = side * N_SIDE * 2
                for j in range(N_SIDE):
                    px = gcol(jbase + 2 * j)
                    py = gcol(jbase + 2 * j + 1)
                    pcx = px - bx
                    pcy = py - by
                    hs.append((pcx * dx + pcy * dy) * scale)
                    rs.append(jnp.abs(pcy * dx - pcx * dy) * inv_s)
                rk = _side_interp(hs, rs, lane, shb, srb, hist)
                if side == 0:
                    for k in range(NUM_SLICES):
                        r1buf[k, :] = rk[k]
                else:
                    for k in range(NUM_SLICES):
                        d = r1buf[k, :] + rk[k]
                        acc = acc + d * d
            outbuf[pl.ds(bi * L, L)] = acc * (_VOL_C * length)
            return c2

        lax.fori_loop(0, NBLK, block_body, 0)
        pltpu.sync_copy(outbuf, out_hbm.at[pl.ds(base, CH)])
        return carry

    lax.fori_loop(0, NCHUNK, chunk_body, 0)


@functools.partial(
    pl.kernel,
    out_type=jax.ShapeDtypeStruct((ROWS,), jnp.float32),
    mesh=plsc.VectorSubcoreMesh(
        core_axis_name="c", subcore_axis_name="s", num_cores=NC, num_subcores=NS
    ),
    scratch_types=[
        pltpu.VMEM((CH * ROW_W,), jnp.float32),
        pltpu.VMEM((N_SIDE, L), jnp.float32),
        pltpu.VMEM((N_SIDE, L), jnp.float32),
        pltpu.VMEM((N_SIDE, L), jnp.int32),
        pltpu.VMEM((NUM_SLICES, L), jnp.float32),
        pltpu.VMEM((CH,), jnp.float32),
    ],
    compiler_params=pltpu.CompilerParams(needs_layout_passes=False),
)
def _sc_volume(pts_hbm, out_hbm, buf, shb, srb, hist, r1buf, outbuf):
    _sc_body(pts_hbm, out_hbm, buf, shb, srb, hist, r1buf, outbuf)


@jax.jit
def kernel(points):
    flat = points.reshape(-1)
    return _sc_volume(flat).reshape(B, T)


# speedup vs baseline: 57.9517x; 44.9585x over previous
import functools

import jax
import jax.numpy as jnp
from jax import lax
from jax.experimental import pallas as pl
from jax.experimental.pallas import tpu as pltpu
from jax.experimental.pallas import tpu_sc as plsc

B, T = 1024, 256
NUM_SLICES = 20
N_SIDE = 21
NPTS = 42
BWORDS = NPTS * 2 * T

NC, NS, L = 2, 16, 16
NW = NC * NS
B_PER_W = B // NW
CB = 2
NCHUNK = B_PER_W // CB
NBLK = CB * (T // L)
OUT_W = B_PER_W * T

import numpy as _np

_TARGETS = _np.linspace(0.05, 0.95, NUM_SLICES, dtype=_np.float32)
_T0 = float(_TARGETS[0])
_INV_STEP = float(1.0 / (_TARGETS[1] - _TARGETS[0]))
_VOL_C = float(3.14159 / 4.0 / NUM_SLICES)


def _oddeven_network(n):
    p2 = 1 << (n - 1).bit_length()
    pairs = []

    def merge(lo, hi, r):
        step = r * 2
        if step < hi - lo:
            merge(lo, hi, step)
            merge(lo + r, hi, step)
            for i in range(lo + r, hi - r, step):
                pairs.append((i, i + r))
        else:
            pairs.append((lo, lo + r))

    def sort(lo, hi):
        if hi - lo >= 2:
            mid = lo + (hi - lo) // 2
            sort(lo, mid)
            sort(mid, hi)
            merge(lo, hi, 1)

    sort(0, p2)
    return [(a, b) for (a, b) in pairs if a < n and b < n]


_NET = _oddeven_network(N_SIDE)


def _side_interp(hs, rs, lane, shb, srb, hist):
    hs = list(hs)
    rs = list(rs)
    for a, b in _NET:
        swap = hs[a] > hs[b]
        ha = jnp.minimum(hs[a], hs[b])
        hb = jnp.maximum(hs[a], hs[b])
        ra = jnp.where(swap, rs[b], rs[a])
        rb = jnp.where(swap, rs[a], rs[b])
        hs[a], hs[b], rs[a], rs[b] = ha, hb, ra, rb

    zero_i = jnp.zeros((L,), jnp.int32)
    one_i = jnp.ones((L,), jnp.int32)
    for j in range(N_SIDE):
        shb[j, :] = hs[j]
        srb[j, :] = rs[j]
        hist[j, :] = zero_i
    for j in range(N_SIDE):
        q = (hs[j] - _T0) * _INV_STEP + 1.0
        q = jnp.minimum(jnp.maximum(q, 0.0), float(NUM_SLICES) + 0.99)
        plsc.addupdate_scatter(hist, [q.astype(jnp.int32), lane], one_i)

    out = []
    cnt = zero_i
    for k in range(NUM_SLICES):
        cnt = cnt + hist[k, :]
        lo = jnp.minimum(jnp.maximum(cnt - 1, 0), N_SIDE - 2)
        hi = lo + 1
        xl = plsc.load_gather(shb, [lo, lane])
        xh = plsc.load_gather(shb, [hi, lane])
        yl = plsc.load_gather(srb, [lo, lane])
        yh = plsc.load_gather(srb, [hi, lane])
        w = (float(_TARGETS[k]) - xl) / (xh - xl + 1e-6)
        w = jnp.minimum(jnp.maximum(w, 0.0), 1.0)
        out.append(yl + w * (yh - yl))
    return out


def _sc_body(pts_hbm, out_hbm, buf, shb, srb, hist, r1buf, outbuf):
    wid = lax.axis_index("s") * NC + lax.axis_index("c")
    b0 = wid * B_PER_W
    lane = lax.iota(jnp.int32, L)

    def chunk_body(ci, carry):
        pltpu.sync_copy(pts_hbm.at[pl.ds((b0 + ci * CB) * BWORDS, CB * BWORDS)], buf)

        def block_body(bi, c2):
            bloc = bi >> 4
            th = (bi >> 3) & 1
            tl0 = (bi & 7) * L
            cbase = bloc * BWORDS + th * 256 + tl0

            def gcol(pt, xy):
                return buf[pl.ds(cbase + pt * 512 + xy * 128, L)]

            p0x, p0y = gcol(0, 0), gcol(0, 1)
            q0x, q0y = gcol(41, 0), gcol(41, 1)
            a0x, a0y = gcol(20, 0), gcol(20, 1)
            a1x, a1y = gcol(21, 0), gcol(21, 1)
            bx = (p0x + q0x) * 0.5
            by = (p0y + q0y) * 0.5
            ax = (a0x + a1x) * 0.5
            ay = (a0y + a1y) * 0.5
            dx = ax - bx
            dy = ay - by
            len2 = jnp.maximum(dx * dx + dy * dy, 1e-20)
            yi = 0x5F3759DF - (plsc.bitcast(len2, jnp.int32) >> 1)
            y = plsc.bitcast(yi, jnp.float32)
            hx = 0.5 * len2
            for _ in range(3):
                y = y * (1.5 - hx * y * y)
            length = len2 * y
            safe = jnp.maximum(length, 1e-6)
            inv_s = 1.0 / safe
            scale = inv_s * inv_s

            acc = jnp.zeros((L,), jnp.float32)
            for side in (0, 1):
                hs = []
                rs = []
                jbase = side * N_SIDE
                for j in range(N_SIDE):
                    px = gcol(jbase + j, 0)
                    py = gcol(jbase + j, 1)
                    pcx = px - bx
                    pcy = py - by
                    hs.append((pcx * dx + pcy * dy) * scale)
                    rs.append(jnp.abs(pcy * dx - pcx * dy) * inv_s)
                rk = _side_interp(hs, rs, lane, shb, srb, hist)
                if side == 0:
                    for k in range(NUM_SLICES):
                        r1buf[k, :] = rk[k]
                else:
                    for k in range(NUM_SLICES):
                        d = r1buf[k, :] + rk[k]
                        acc = acc + d * d
            b_local = ci * CB + bloc
            out_off = (((b_local >> 3) << 1) + th) * 1024 + (b_local & 7) * 128 + tl0
            outbuf[pl.ds(out_off, L)] = acc * (_VOL_C * length)
            return c2

        lax.fori_loop(0, NBLK, block_body, 0)
        return carry

    lax.fori_loop(0, NCHUNK, chunk_body, 0)
    pltpu.sync_copy(outbuf, out_hbm.at[pl.ds(wid * OUT_W, OUT_W)])


@functools.partial(
    pl.kernel,
    out_type=jax.ShapeDtypeStruct((B * T,), jnp.float32),
    mesh=plsc.VectorSubcoreMesh(
        core_axis_name="c", subcore_axis_name="s", num_cores=NC, num_subcores=NS
    ),
    scratch_types=[
        pltpu.VMEM((CB * BWORDS,), jnp.float32),
        pltpu.VMEM((N_SIDE, L), jnp.float32),
        pltpu.VMEM((N_SIDE, L), jnp.float32),
        pltpu.VMEM((N_SIDE, L), jnp.int32),
        pltpu.VMEM((NUM_SLICES, L), jnp.float32),
        pltpu.VMEM((OUT_W,), jnp.float32),
    ],
    compiler_params=pltpu.CompilerParams(needs_layout_passes=False),
)
def _sc_volume(pts_hbm, out_hbm, buf, shb, srb, hist, r1buf, outbuf):
    _sc_body(pts_hbm, out_hbm, buf, shb, srb, hist, r1buf, outbuf)


@jax.jit
def kernel(points):
    flat = (
        points.reshape(B, 2, 128, NPTS, 2)
        .transpose(0, 3, 1, 4, 2)
        .reshape(-1)
    )
    out = _sc_volume(flat)
    return out.reshape(B // 8, T // 128, 8, 128).transpose(0, 2, 1, 3).reshape(B, T)
